# baseline (device time: 287762 ns/iter reference)
import functools

import jax
import jax.numpy as jnp
from jax import lax
from jax.experimental import pallas as pl
from jax.experimental.pallas import tpu as pltpu

N_DEV = 4
SQ = 1024
HQ = 8
DH = 128
SCALE = 0.08838834764831843


def kernel(x, Wq, K_ext, V_ext, Wo):
    def body(x_ref, wq_ref, k_ref, v_ref, wo_ref, out_ref,
             kbuf, vbuf, gbuf,
             local_sems, scat_send, scat_recv, ring_send, ring_recv):
        my = lax.axis_index("i")
        right = lax.rem(my + 1, N_DEV)

        barrier_sem = pltpu.get_barrier_semaphore()
        for off in (1, 2, 3):
            pl.semaphore_signal(
                barrier_sem, inc=1,
                device_id=(lax.rem(my + off, N_DEV),),
                device_id_type=pl.DeviceIdType.MESH,
            )
        pl.semaphore_wait(barrier_sem, 3)

        @pl.when(my == 0)
        def _():
            ck = pltpu.make_async_copy(
                k_ref.at[0, :, pl.ds(0, HQ), :], kbuf, local_sems.at[0])
            cv = pltpu.make_async_copy(
                v_ref.at[0, :, pl.ds(0, HQ), :], vbuf, local_sems.at[1])
            ck.start()
            cv.start()
            for p in (1, 2, 3):
                rk = pltpu.make_async_remote_copy(
                    src_ref=k_ref.at[0, :, pl.ds(HQ * p, HQ), :],
                    dst_ref=kbuf,
                    send_sem=scat_send.at[2 * (p - 1)],
                    recv_sem=scat_recv.at[0],
                    device_id=(p,),
                    device_id_type=pl.DeviceIdType.MESH,
                )
                rv = pltpu.make_async_remote_copy(
                    src_ref=v_ref.at[0, :, pl.ds(HQ * p, HQ), :],
                    dst_ref=vbuf,
                    send_sem=scat_send.at[2 * (p - 1) + 1],
                    recv_sem=scat_recv.at[1],
                    device_id=(p,),
                    device_id_type=pl.DeviceIdType.MESH,
                )
                rk.start()
                rv.start()

        x_bf = x_ref[0].astype(jnp.bfloat16)
        q = jnp.dot(x_bf, wq_ref[:].astype(jnp.bfloat16),
                    preferred_element_type=jnp.float32)
        q_bf = q.astype(jnp.bfloat16)

        @pl.when(my == 0)
        def _():
            pltpu.make_async_copy(
                k_ref.at[0, :, pl.ds(0, HQ), :], kbuf, local_sems.at[0]).wait()
            pltpu.make_async_copy(
                v_ref.at[0, :, pl.ds(0, HQ), :], vbuf, local_sems.at[1]).wait()

        @pl.when(my != 0)
        def _():
            rk = pltpu.make_async_remote_copy(
                src_ref=k_ref.at[0, :, pl.ds(0, HQ), :],
                dst_ref=kbuf,
                send_sem=scat_send.at[0],
                recv_sem=scat_recv.at[0],
                device_id=(0,),
                device_id_type=pl.DeviceIdType.MESH,
            )
            rv = pltpu.make_async_remote_copy(
                src_ref=v_ref.at[0, :, pl.ds(0, HQ), :],
                dst_ref=vbuf,
                send_sem=scat_send.at[1],
                recv_sem=scat_recv.at[1],
                device_id=(0,),
                device_id_type=pl.DeviceIdType.MESH,
            )
            rk.wait_recv()
            rv.wait_recv()

        rows = lax.broadcasted_iota(jnp.int32, (SQ, SQ), 0) // 64
        cols = lax.broadcasted_iota(jnp.int32, (SQ, SQ), 1) // 64
        mask = cols <= rows

        ctx_cols = []
        for h in range(HQ):
            qh = q_bf[:, DH * h:DH * (h + 1)]
            kh = kbuf[:, h, :].astype(jnp.bfloat16)
            s = lax.dot_general(
                qh, kh, (((1,), (1,)), ((), ())),
                preferred_element_type=jnp.float32) * SCALE
            s = jnp.where(mask, s, -1e9)
            s = s - jnp.max(s, axis=1, keepdims=True)
            w = jnp.exp(s)
            w = w / jnp.sum(w, axis=1, keepdims=True)
            vh = vbuf[:, h, :].astype(jnp.bfloat16)
            ctx_cols.append(jnp.dot(w.astype(jnp.bfloat16), vh,
                                    preferred_element_type=jnp.float32))
        ctx = jnp.concatenate(ctx_cols, axis=1)

        partial = jnp.dot(ctx.astype(jnp.bfloat16),
                          wo_ref[:].astype(jnp.bfloat16),
                          preferred_element_type=jnp.float32)

        @pl.when(my == 0)
        def _():
            for p in (1, 2, 3):
                pltpu.make_async_remote_copy(
                    src_ref=k_ref.at[0, :, pl.ds(HQ * p, HQ), :],
                    dst_ref=kbuf,
                    send_sem=scat_send.at[2 * (p - 1)],
                    recv_sem=scat_recv.at[0],
                    device_id=(p,),
                    device_id_type=pl.DeviceIdType.MESH,
                ).wait_send()
                pltpu.make_async_remote_copy(
                    src_ref=v_ref.at[0, :, pl.ds(HQ * p, HQ), :],
                    dst_ref=vbuf,
                    send_sem=scat_send.at[2 * (p - 1) + 1],
                    recv_sem=scat_recv.at[1],
                    device_id=(p,),
                    device_id_type=pl.DeviceIdType.MESH,
                ).wait_send()

        gbuf[0] = partial.astype(jnp.bfloat16)
        acc = partial
        for h in range(N_DEV - 1):
            rdma = pltpu.make_async_remote_copy(
                src_ref=gbuf.at[h],
                dst_ref=gbuf.at[h + 1],
                send_sem=ring_send.at[h],
                recv_sem=ring_recv.at[h],
                device_id=(right,),
                device_id_type=pl.DeviceIdType.MESH,
            )
            rdma.start()
            rdma.wait()
            acc = acc + gbuf[h + 1].astype(jnp.float32)

        out_ref[0] = acc

    return pl.pallas_call(
        body,
        out_shape=jax.ShapeDtypeStruct((1, SQ, 1024), jnp.float32),
        in_specs=[
            pl.BlockSpec(memory_space=pltpu.VMEM),
            pl.BlockSpec(memory_space=pltpu.VMEM),
            pl.BlockSpec(memory_space=pl.ANY),
            pl.BlockSpec(memory_space=pl.ANY),
            pl.BlockSpec(memory_space=pltpu.VMEM),
        ],
        out_specs=pl.BlockSpec(memory_space=pltpu.VMEM),
        scratch_shapes=[
            pltpu.VMEM((SQ, HQ, DH), jnp.float32),
            pltpu.VMEM((SQ, HQ, DH), jnp.float32),
            pltpu.VMEM((N_DEV, SQ, 1024), jnp.bfloat16),
            pltpu.SemaphoreType.DMA((2,)),
            pltpu.SemaphoreType.DMA((6,)),
            pltpu.SemaphoreType.DMA((2,)),
            pltpu.SemaphoreType.DMA((3,)),
            pltpu.SemaphoreType.DMA((3,)),
        ],
        compiler_params=pltpu.CompilerParams(collective_id=0),
    )(x, Wq, K_ext, V_ext, Wo)


# device time: 179369 ns/iter; 1.6043x vs baseline; 1.6043x over previous
import jax
import jax.numpy as jnp
from jax import lax
from jax.experimental import pallas as pl
from jax.experimental.pallas import tpu as pltpu

N_DEV = 4
SQ = 1024
HQ = 8
DH = 128
BLK = 64
SCALE = 0.08838834764831843
NEG = -1e4


def kernel(x, Wq, K_ext, V_ext, Wo):
    def body(x_ref, wq_ref, k_ref, v_ref, wo_ref, out_ref,
             tmpk, tmpv, kstage, vstage, kbuf, vbuf, ctx_buf, sbuf, rbuf,
             local_sems, scat_send, scat_recv, ar_send, ar_recv):
        my = lax.axis_index("i")

        barrier_sem = pltpu.get_barrier_semaphore()
        for off in (1, 2, 3):
            pl.semaphore_signal(
                barrier_sem, inc=1,
                device_id=(lax.rem(my + off, N_DEV),),
                device_id_type=pl.DeviceIdType.MESH,
            )
        pl.semaphore_wait(barrier_sem, 3)

        @pl.when(my == 0)
        def _():
            for p in (2, 1, 3, 0):
                ck = pltpu.make_async_copy(
                    k_ref.at[0, :, pl.ds(HQ * p, HQ), :], tmpk,
                    local_sems.at[0])
                cv = pltpu.make_async_copy(
                    v_ref.at[0, :, pl.ds(HQ * p, HQ), :], tmpv,
                    local_sems.at[1])
                ck.start()
                cv.start()
                ck.wait()
                cv.wait()
                if p != 0:
                    kstage[p - 1] = tmpk[:].astype(jnp.bfloat16)
                    vstage[p - 1] = tmpv[:].astype(jnp.bfloat16)
                    pltpu.make_async_remote_copy(
                        src_ref=kstage.at[p - 1], dst_ref=kbuf,
                        send_sem=scat_send.at[2 * (p - 1)],
                        recv_sem=scat_recv.at[0],
                        device_id=(p,),
                        device_id_type=pl.DeviceIdType.MESH,
                    ).start()
                    pltpu.make_async_remote_copy(
                        src_ref=vstage.at[p - 1], dst_ref=vbuf,
                        send_sem=scat_send.at[2 * (p - 1) + 1],
                        recv_sem=scat_recv.at[1],
                        device_id=(p,),
                        device_id_type=pl.DeviceIdType.MESH,
                    ).start()
                else:
                    kbuf[:] = tmpk[:].astype(jnp.bfloat16)
                    vbuf[:] = tmpv[:].astype(jnp.bfloat16)

        x_bf = x_ref[0].astype(jnp.bfloat16)
        q = jnp.dot(x_bf, wq_ref[:].astype(jnp.bfloat16),
                    preferred_element_type=jnp.float32)
        q_bf = (q * SCALE).astype(jnp.bfloat16)

        @pl.when(my != 0)
        def _():
            rk = pltpu.make_async_remote_copy(
                src_ref=kstage.at[0], dst_ref=kbuf,
                send_sem=scat_send.at[0], recv_sem=scat_recv.at[0],
                device_id=(0,), device_id_type=pl.DeviceIdType.MESH)
            rv = pltpu.make_async_remote_copy(
                src_ref=vstage.at[0], dst_ref=vbuf,
                send_sem=scat_send.at[1], recv_sem=scat_recv.at[1],
                device_id=(0,), device_id_type=pl.DeviceIdType.MESH)
            rk.wait_recv()
            rv.wait_recv()

        H = SQ // 2

        def mk_mask(nrows, ncols, row0):
            r = lax.broadcasted_iota(jnp.int32, (nrows, ncols), 0) // BLK
            c = lax.broadcasted_iota(jnp.int32, (nrows, ncols), 1) // BLK
            return c <= (r + row0 // BLK)

        mask0 = mk_mask(H, H, 0)
        mask1 = mk_mask(H, SQ, H)

        for r0, nkv, mask in ((0, H, mask0), (H, SQ, mask1)):
            for h in range(HQ):
                qh = q_bf[r0:r0 + H, DH * h:DH * (h + 1)]
                kh = kbuf[:nkv, h, :]
                s = lax.dot_general(
                    qh, kh, (((1,), (1,)), ((), ())),
                    preferred_element_type=jnp.float32)
                w = jnp.exp(jnp.where(mask, s, NEG))
                w = (w / jnp.sum(w, axis=1, keepdims=True)).astype(jnp.bfloat16)
                ctx_buf[r0:r0 + H, DH * h:DH * (h + 1)] = jnp.dot(
                    w, vbuf[:nkv, h, :],
                    preferred_element_type=jnp.float32).astype(jnp.bfloat16)

        partial = jnp.dot(ctx_buf[:], wo_ref[:].astype(jnp.bfloat16),
                          preferred_element_type=jnp.float32)

        @pl.when(my == 0)
        def _():
            for p in (1, 2, 3):
                pltpu.make_async_remote_copy(
                    src_ref=kstage.at[p - 1], dst_ref=kbuf,
                    send_sem=scat_send.at[2 * (p - 1)],
                    recv_sem=scat_recv.at[0],
                    device_id=(p,),
                    device_id_type=pl.DeviceIdType.MESH,
                ).wait_send()
                pltpu.make_async_remote_copy(
                    src_ref=vstage.at[p - 1], dst_ref=vbuf,
                    send_sem=scat_send.at[2 * (p - 1) + 1],
                    recv_sem=scat_recv.at[1],
                    device_id=(p,),
                    device_id_type=pl.DeviceIdType.MESH,
                ).wait_send()

        sbuf[:] = partial.astype(jnp.bfloat16)
        p1 = jnp.bitwise_xor(my, 1)
        r1 = pltpu.make_async_remote_copy(
            src_ref=sbuf, dst_ref=rbuf.at[0],
            send_sem=ar_send.at[0], recv_sem=ar_recv.at[0],
            device_id=(p1,), device_id_type=pl.DeviceIdType.MESH)
        r1.start()
        r1.wait()
        acc = partial + rbuf[0].astype(jnp.float32)

        sbuf[:] = acc.astype(jnp.bfloat16)
        p2 = (N_DEV - 1) - my
        r2 = pltpu.make_async_remote_copy(
            src_ref=sbuf, dst_ref=rbuf.at[1],
            send_sem=ar_send.at[1], recv_sem=ar_recv.at[1],
            device_id=(p2,), device_id_type=pl.DeviceIdType.MESH)
        r2.start()
        r2.wait()
        out_ref[0] = acc + rbuf[1].astype(jnp.float32)

    return pl.pallas_call(
        body,
        out_shape=jax.ShapeDtypeStruct((1, SQ, 1024), jnp.float32),
        in_specs=[
            pl.BlockSpec(memory_space=pltpu.VMEM),
            pl.BlockSpec(memory_space=pltpu.VMEM),
            pl.BlockSpec(memory_space=pl.ANY),
            pl.BlockSpec(memory_space=pl.ANY),
            pl.BlockSpec(memory_space=pltpu.VMEM),
        ],
        out_specs=pl.BlockSpec(memory_space=pltpu.VMEM),
        scratch_shapes=[
            pltpu.VMEM((SQ, HQ, DH), jnp.float32),
            pltpu.VMEM((SQ, HQ, DH), jnp.float32),
            pltpu.VMEM((3, SQ, HQ, DH), jnp.bfloat16),
            pltpu.VMEM((3, SQ, HQ, DH), jnp.bfloat16),
            pltpu.VMEM((SQ, HQ, DH), jnp.bfloat16),
            pltpu.VMEM((SQ, HQ, DH), jnp.bfloat16),
            pltpu.VMEM((SQ, HQ * DH), jnp.bfloat16),
            pltpu.VMEM((SQ, 1024), jnp.bfloat16),
            pltpu.VMEM((2, SQ, 1024), jnp.bfloat16),
            pltpu.SemaphoreType.DMA((2,)),
            pltpu.SemaphoreType.DMA((6,)),
            pltpu.SemaphoreType.DMA((2,)),
            pltpu.SemaphoreType.DMA((2,)),
            pltpu.SemaphoreType.DMA((2,)),
        ],
        compiler_params=pltpu.CompilerParams(
            collective_id=0, vmem_limit_bytes=110 * 1024 * 1024),
    )(x, Wq, K_ext, V_ext, Wo)


# device time: 151500 ns/iter; 1.8994x vs baseline; 1.1840x over previous
import jax
import jax.numpy as jnp
from jax import lax
from jax.experimental import pallas as pl
from jax.experimental.pallas import tpu as pltpu

N_DEV = 4
SQ = 1024
HQ = 8
HH = HQ // 2
DH = 128
BLK = 64
H = SQ // 2
SCALE = 0.08838834764831843
NEG = -1e4


def kernel(x, Wq, K_ext, V_ext, Wo):
    def body(x_ref, wq_ref, k_ref, v_ref, wo_ref, out_ref,
             tmpk, tmpv, kstage, vstage, kbuf, vbuf, ctx_buf, sbuf, rbuf,
             local_sems, scat_send, krecv, vrecv, ar_send, ar_recv):
        my = lax.axis_index("i")

        barrier_sem = pltpu.get_barrier_semaphore()
        for off in (1, 2, 3):
            pl.semaphore_signal(
                barrier_sem, inc=1,
                device_id=(lax.rem(my + off, N_DEV),),
                device_id_type=pl.DeviceIdType.MESH,
            )
        pl.semaphore_wait(barrier_sem, 3)

        def scat_chunk(j, p, half, tensor):
            stage, buf, sems = (
                (kstage, kbuf, krecv) if tensor == 0 else (vstage, vbuf, vrecv))
            return pltpu.make_async_remote_copy(
                src_ref=stage.at[j, :, pl.ds(HH * half, HH), :],
                dst_ref=buf.at[:, pl.ds(HH * half, HH), :],
                send_sem=scat_send.at[4 * j + 2 * half + tensor],
                recv_sem=sems.at[half],
                device_id=(p,),
                device_id_type=pl.DeviceIdType.MESH,
            )

        @pl.when(my == 0)
        def _():
            for j, p in enumerate((2, 1, 3, 0)):
                ck = pltpu.make_async_copy(
                    k_ref.at[0, :, pl.ds(HQ * p, HQ), :], tmpk,
                    local_sems.at[0])
                cv = pltpu.make_async_copy(
                    v_ref.at[0, :, pl.ds(HQ * p, HQ), :], tmpv,
                    local_sems.at[1])
                ck.start()
                cv.start()
                ck.wait()
                cv.wait()
                if p != 0:
                    kstage[j] = tmpk[:].astype(jnp.bfloat16)
                    vstage[j] = tmpv[:].astype(jnp.bfloat16)
                    for half in (0, 1):
                        scat_chunk(j, p, half, 0).start()
                        scat_chunk(j, p, half, 1).start()
                else:
                    kbuf[:] = tmpk[:].astype(jnp.bfloat16)
                    vbuf[:] = tmpv[:].astype(jnp.bfloat16)

        x_bf = x_ref[0].astype(jnp.bfloat16)
        q = jnp.dot(x_bf, wq_ref[:].astype(jnp.bfloat16),
                    preferred_element_type=jnp.float32)
        q_bf = (q * SCALE).astype(jnp.bfloat16)
        wo_bf = wo_ref[:].astype(jnp.bfloat16)

        def mk_mask(nrows, ncols, row0):
            r = lax.broadcasted_iota(jnp.int32, (nrows, ncols), 0) // BLK
            c = lax.broadcasted_iota(jnp.int32, (nrows, ncols), 1) // BLK
            return c <= (r + row0 // BLK)

        mask0 = mk_mask(H, H, 0)
        mask1 = mk_mask(H, SQ, H)

        def attn(r0, nkv, mask, h):
            qh = q_bf[r0:r0 + H, DH * h:DH * (h + 1)]
            s = lax.dot_general(
                qh, kbuf[:nkv, h, :], (((1,), (1,)), ((), ())),
                preferred_element_type=jnp.float32)
            w = jnp.exp(jnp.where(mask, s, NEG))
            w = (w / jnp.sum(w, axis=1, keepdims=True)).astype(jnp.bfloat16)
            ctx_buf[r0:r0 + H, DH * h:DH * (h + 1)] = jnp.dot(
                w, vbuf[:nkv, h, :],
                preferred_element_type=jnp.float32).astype(jnp.bfloat16)

        def wait_half(half):
            @pl.when(my != 0)
            def _():
                scat_chunk(0, 0, half, 0).wait_recv()
                scat_chunk(0, 0, half, 1).wait_recv()

        wait_half(0)
        for h in range(HH):
            attn(0, H, mask0, h)
            attn(H, SQ, mask1, h)
        wait_half(1)
        for h in range(HH, HQ):
            attn(0, H, mask0, h)

        p1 = jnp.bitwise_xor(my, 1)
        p2 = (N_DEV - 1) - my

        def xchg(slot, partner):
            return pltpu.make_async_remote_copy(
                src_ref=sbuf.at[slot], dst_ref=rbuf.at[slot],
                send_sem=ar_send.at[slot], recv_sem=ar_recv.at[slot],
                device_id=(partner,), device_id_type=pl.DeviceIdType.MESH)

        partial0 = jnp.dot(ctx_buf[0:H], wo_bf,
                           preferred_element_type=jnp.float32)
        sbuf[0] = partial0.astype(jnp.bfloat16)
        ar1a = xchg(0, p1)
        ar1a.start()

        for h in range(HH, HQ):
            attn(H, SQ, mask1, h)

        partial1 = jnp.dot(ctx_buf[H:SQ], wo_bf,
                           preferred_element_type=jnp.float32)
        sbuf[1] = partial1.astype(jnp.bfloat16)
        ar1b = xchg(1, p1)
        ar1b.start()

        @pl.when(my == 0)
        def _():
            for j, p in enumerate((2, 1, 3)):
                for half in (0, 1):
                    scat_chunk(j, p, half, 0).wait_send()
                    scat_chunk(j, p, half, 1).wait_send()

        ar1a.wait()
        acc0 = partial0 + rbuf[0].astype(jnp.float32)
        sbuf[2] = acc0.astype(jnp.bfloat16)
        ar2a = xchg(2, p2)
        ar2a.start()

        ar1b.wait()
        acc1 = partial1 + rbuf[1].astype(jnp.float32)
        sbuf[3] = acc1.astype(jnp.bfloat16)
        ar2b = xchg(3, p2)
        ar2b.start()

        ar2a.wait()
        out_ref[0, 0:H] = acc0 + rbuf[2].astype(jnp.float32)
        ar2b.wait()
        out_ref[0, H:SQ] = acc1 + rbuf[3].astype(jnp.float32)

    return pl.pallas_call(
        body,
        out_shape=jax.ShapeDtypeStruct((1, SQ, 1024), jnp.float32),
        in_specs=[
            pl.BlockSpec(memory_space=pltpu.VMEM),
            pl.BlockSpec(memory_space=pltpu.VMEM),
            pl.BlockSpec(memory_space=pl.ANY),
            pl.BlockSpec(memory_space=pl.ANY),
            pl.BlockSpec(memory_space=pltpu.VMEM),
        ],
        out_specs=pl.BlockSpec(memory_space=pltpu.VMEM),
        scratch_shapes=[
            pltpu.VMEM((SQ, HQ, DH), jnp.float32),
            pltpu.VMEM((SQ, HQ, DH), jnp.float32),
            pltpu.VMEM((3, SQ, HQ, DH), jnp.bfloat16),
            pltpu.VMEM((3, SQ, HQ, DH), jnp.bfloat16),
            pltpu.VMEM((SQ, HQ, DH), jnp.bfloat16),
            pltpu.VMEM((SQ, HQ, DH), jnp.bfloat16),
            pltpu.VMEM((SQ, HQ * DH), jnp.bfloat16),
            pltpu.VMEM((4, H, 1024), jnp.bfloat16),
            pltpu.VMEM((4, H, 1024), jnp.bfloat16),
            pltpu.SemaphoreType.DMA((2,)),
            pltpu.SemaphoreType.DMA((12,)),
            pltpu.SemaphoreType.DMA((2,)),
            pltpu.SemaphoreType.DMA((2,)),
            pltpu.SemaphoreType.DMA((4,)),
            pltpu.SemaphoreType.DMA((4,)),
        ],
        compiler_params=pltpu.CompilerParams(
            collective_id=0, vmem_limit_bytes=110 * 1024 * 1024),
    )(x, Wq, K_ext, V_ext, Wo)


# device time: 125051 ns/iter; 2.3012x vs baseline; 1.2115x over previous
import jax
import jax.numpy as jnp
from jax import lax
from jax.experimental import pallas as pl
from jax.experimental.pallas import tpu as pltpu

N_DEV = 4
SQ = 1024
HQ = 8
HH = HQ // 2
DH = 128
BLK = 64
H = SQ // 2
SCALE = 0.08838834764831843
NEG = -1e4


def kernel(x, Wq, K_ext, V_ext, Wo):
    def body(x_ref, wq_ref, k_ref, v_ref, wo_ref, out_ref,
             tmp, kstage, vstage, kbuf, vbuf, relayk, relayv,
             ctx_buf, sbuf, rbuf,
             local_sems, scat_send, krecv, vrecv, relay_recv, fwd_send,
             ar_send, ar_recv):
        my = lax.axis_index("i")

        ORDER = (2, 1, 3, 0)
        SEQ = tuple((p, t) for p in ORDER for t in (0, 1))

        def dma_one(i, slot):
            p, t = SEQ[i]
            src_ref = (k_ref if t == 0 else v_ref).at[0, :, pl.ds(HQ * p, HQ), :]
            return pltpu.make_async_copy(src_ref, tmp.at[slot],
                                         local_sems.at[slot])

        @pl.when(my == 0)
        def _():
            dma_one(0, 0).start()

        barrier_sem = pltpu.get_barrier_semaphore()
        for off in (1, 2, 3):
            pl.semaphore_signal(
                barrier_sem, inc=1,
                device_id=(lax.rem(my + off, N_DEV),),
                device_id_type=pl.DeviceIdType.MESH,
            )
        pl.semaphore_wait(barrier_sem, 3)

        JOF = {2: 0, 1: 1, 3: 2}

        def scat_chunk(p, half, tensor):
            j = JOF[p]
            stage = kstage if tensor == 0 else vstage
            src = stage.at[j, :, pl.ds(HH * half, HH), :]
            send = scat_send.at[4 * j + 2 * half + tensor]
            if p == 2:
                target = 1 if half == 0 else 3
                dst = relayk if tensor == 0 else relayv
                recv = relay_recv.at[tensor]
            else:
                target = p
                buf, sems = (kbuf, krecv) if tensor == 0 else (vbuf, vrecv)
                dst = buf.at[:, pl.ds(HH * half, HH), :]
                recv = sems.at[half]
            return pltpu.make_async_remote_copy(
                src_ref=src, dst_ref=dst, send_sem=send, recv_sem=recv,
                device_id=(target,), device_id_type=pl.DeviceIdType.MESH)

        @pl.when(my == 0)
        def _():
            for i, (p, t) in enumerate(SEQ):
                slot = i % 2
                dma_one(i, slot).wait()
                if i + 1 < len(SEQ):
                    dma_one(i + 1, 1 - slot).start()
                if p != 0:
                    stage = kstage if t == 0 else vstage
                    stage[JOF[p]] = tmp[slot].astype(jnp.bfloat16)
                    scat_chunk(p, 0, t).start()
                    scat_chunk(p, 1, t).start()
                else:
                    buf = kbuf if t == 0 else vbuf
                    buf[:] = tmp[slot].astype(jnp.bfloat16)

        def fwd_desc(half, tensor):
            src = relayk if tensor == 0 else relayv
            buf, sems = (kbuf, krecv) if tensor == 0 else (vbuf, vrecv)
            return pltpu.make_async_remote_copy(
                src_ref=src, dst_ref=buf.at[:, pl.ds(HH * half, HH), :],
                send_sem=fwd_send.at[tensor], recv_sem=sems.at[half],
                device_id=(2,), device_id_type=pl.DeviceIdType.MESH)

        def do_relay(half):
            for tensor in (0, 1):
                scat_chunk(2, half, tensor).wait_recv()
                fwd_desc(half, tensor).start()

        @pl.when(my == 1)
        def _():
            do_relay(0)

        @pl.when(my == 3)
        def _():
            do_relay(1)

        x_bf = x_ref[0].astype(jnp.bfloat16)
        q = jnp.dot(x_bf, wq_ref[:].astype(jnp.bfloat16),
                    preferred_element_type=jnp.float32)
        q_bf = (q * SCALE).astype(jnp.bfloat16)
        wo_bf = wo_ref[:].astype(jnp.bfloat16)

        def mk_mask(nrows, ncols, row0):
            r = lax.broadcasted_iota(jnp.int32, (nrows, ncols), 0) // BLK
            c = lax.broadcasted_iota(jnp.int32, (nrows, ncols), 1) // BLK
            return c <= (r + row0 // BLK)

        mask0 = mk_mask(H, H, 0)
        mask1 = mk_mask(H, SQ, H)

        def attn(r0, nkv, mask, h):
            qh = q_bf[r0:r0 + H, DH * h:DH * (h + 1)]
            s = lax.dot_general(
                qh, kbuf[:nkv, h, :], (((1,), (1,)), ((), ())),
                preferred_element_type=jnp.float32)
            w = jnp.exp(jnp.where(mask, s, NEG))
            w = (w / jnp.sum(w, axis=1, keepdims=True)).astype(jnp.bfloat16)
            ctx_buf[r0:r0 + H, DH * h:DH * (h + 1)] = jnp.dot(
                w, vbuf[:nkv, h, :],
                preferred_element_type=jnp.float32).astype(jnp.bfloat16)

        def wait_half(half):
            @pl.when(my != 0)
            def _():
                scat_chunk(1, half, 0).wait_recv()
                scat_chunk(1, half, 1).wait_recv()

        wait_half(0)
        for h in range(HH):
            attn(0, H, mask0, h)
            attn(H, SQ, mask1, h)
        wait_half(1)
        for h in range(HH, HQ):
            attn(0, H, mask0, h)

        p1 = jnp.bitwise_xor(my, 1)
        p2 = (N_DEV - 1) - my

        def xchg(slot, partner):
            return pltpu.make_async_remote_copy(
                src_ref=sbuf.at[slot], dst_ref=rbuf.at[slot],
                send_sem=ar_send.at[slot], recv_sem=ar_recv.at[slot],
                device_id=(partner,), device_id_type=pl.DeviceIdType.MESH)

        partial0 = jnp.dot(ctx_buf[0:H], wo_bf,
                           preferred_element_type=jnp.float32)
        sbuf[0] = partial0.astype(jnp.bfloat16)
        ar1a = xchg(0, p1)
        ar1a.start()

        for h in range(HH, HQ):
            attn(H, SQ, mask1, h)

        partial1 = jnp.dot(ctx_buf[H:SQ], wo_bf,
                           preferred_element_type=jnp.float32)
        sbuf[1] = partial1.astype(jnp.bfloat16)
        ar1b = xchg(1, p1)
        ar1b.start()

        @pl.when(my == 0)
        def _():
            for p in (2, 1, 3):
                for half in (0, 1):
                    scat_chunk(p, half, 0).wait_send()
                    scat_chunk(p, half, 1).wait_send()

        @pl.when(my == 1)
        def _():
            fwd_desc(0, 0).wait_send()
            fwd_desc(0, 1).wait_send()

        @pl.when(my == 3)
        def _():
            fwd_desc(1, 0).wait_send()
            fwd_desc(1, 1).wait_send()

        ar1a.wait()
        acc0 = partial0 + rbuf[0].astype(jnp.float32)
        sbuf[2] = acc0.astype(jnp.bfloat16)
        ar2a = xchg(2, p2)
        ar2a.start()

        ar1b.wait()
        acc1 = partial1 + rbuf[1].astype(jnp.float32)
        sbuf[3] = acc1.astype(jnp.bfloat16)
        ar2b = xchg(3, p2)
        ar2b.start()

        ar2a.wait()
        out_ref[0, 0:H] = acc0 + rbuf[2].astype(jnp.float32)
        ar2b.wait()
        out_ref[0, H:SQ] = acc1 + rbuf[3].astype(jnp.float32)

    return pl.pallas_call(
        body,
        out_shape=jax.ShapeDtypeStruct((1, SQ, 1024), jnp.float32),
        in_specs=[
            pl.BlockSpec(memory_space=pltpu.VMEM),
            pl.BlockSpec(memory_space=pltpu.VMEM),
            pl.BlockSpec(memory_space=pl.ANY),
            pl.BlockSpec(memory_space=pl.ANY),
            pl.BlockSpec(memory_space=pltpu.VMEM),
        ],
        out_specs=pl.BlockSpec(memory_space=pltpu.VMEM),
        scratch_shapes=[
            pltpu.VMEM((2, SQ, HQ, DH), jnp.float32),
            pltpu.VMEM((3, SQ, HQ, DH), jnp.bfloat16),
            pltpu.VMEM((3, SQ, HQ, DH), jnp.bfloat16),
            pltpu.VMEM((SQ, HQ, DH), jnp.bfloat16),
            pltpu.VMEM((SQ, HQ, DH), jnp.bfloat16),
            pltpu.VMEM((SQ, HH, DH), jnp.bfloat16),
            pltpu.VMEM((SQ, HH, DH), jnp.bfloat16),
            pltpu.VMEM((SQ, HQ * DH), jnp.bfloat16),
            pltpu.VMEM((4, H, 1024), jnp.bfloat16),
            pltpu.VMEM((4, H, 1024), jnp.bfloat16),
            pltpu.SemaphoreType.DMA((2,)),
            pltpu.SemaphoreType.DMA((12,)),
            pltpu.SemaphoreType.DMA((2,)),
            pltpu.SemaphoreType.DMA((2,)),
            pltpu.SemaphoreType.DMA((2,)),
            pltpu.SemaphoreType.DMA((2,)),
            pltpu.SemaphoreType.DMA((4,)),
            pltpu.SemaphoreType.DMA((4,)),
        ],
        compiler_params=pltpu.CompilerParams(
            collective_id=0, vmem_limit_bytes=110 * 1024 * 1024),
    )(x, Wq, K_ext, V_ext, Wo)


# device time: 122808 ns/iter; 2.3432x vs baseline; 1.0183x over previous
import jax
import jax.numpy as jnp
from jax import lax
from jax.experimental import pallas as pl
from jax.experimental.pallas import tpu as pltpu

N_DEV = 4
SQ = 1024
HQ = 8
HH = HQ // 2
DH = 128
BLK = 64
H = SQ // 2
SCALE = 0.08838834764831843
NEG = -1e4


def kernel(x, Wq, K_ext, V_ext, Wo):
    def body(x_ref, wq_ref, k_ref, v_ref, wo_ref, out_ref,
             tmp, kstage, vstage, kbuf, vbuf, relayk, relayv,
             ctx_buf, sbuf, rbuf,
             local_sems, scat_send, krecv, vrecv, relay_recv, fwd_send,
             ar_send, ar_recv):
        my = lax.axis_index("i")

        ORDER = (2, 1, 3, 0)
        SEQ = tuple((p, t) for p in ORDER for t in (0, 1))

        def dma_one(i, slot):
            p, t = SEQ[i]
            src_ref = (k_ref if t == 0 else v_ref).at[0, :, pl.ds(HQ * p, HQ), :]
            return pltpu.make_async_copy(src_ref, tmp.at[slot],
                                         local_sems.at[slot])

        @pl.when(my == 0)
        def _():
            dma_one(0, 0).start()

        barrier_sem = pltpu.get_barrier_semaphore()
        for off in (1, 2, 3):
            pl.semaphore_signal(
                barrier_sem, inc=1,
                device_id=(lax.rem(my + off, N_DEV),),
                device_id_type=pl.DeviceIdType.MESH,
            )
        pl.semaphore_wait(barrier_sem, 3)

        JOF = {2: 0, 1: 1, 3: 2}

        def scat_chunk(p, half, tensor):
            j = JOF[p]
            stage = kstage if tensor == 0 else vstage
            src = stage.at[j, :, pl.ds(HH * half, HH), :]
            send = scat_send.at[4 * j + 2 * half + tensor]
            if p == 2:
                target = 1 if half == 0 else 3
                dst = relayk if tensor == 0 else relayv
                recv = relay_recv.at[tensor]
            else:
                target = p
                buf, sems = (kbuf, krecv) if tensor == 0 else (vbuf, vrecv)
                dst = buf.at[:, pl.ds(HH * half, HH), :]
                recv = sems.at[half]
            return pltpu.make_async_remote_copy(
                src_ref=src, dst_ref=dst, send_sem=send, recv_sem=recv,
                device_id=(target,), device_id_type=pl.DeviceIdType.MESH)

        @pl.when(my == 0)
        def _():
            for i, (p, t) in enumerate(SEQ):
                slot = i % 2
                dma_one(i, slot).wait()
                if i + 1 < len(SEQ):
                    dma_one(i + 1, 1 - slot).start()
                if p != 0:
                    stage = kstage if t == 0 else vstage
                    stage[JOF[p]] = tmp[slot].astype(jnp.bfloat16)
                    if t == 0:
                        scat_chunk(p, 0, 0).start()
                    else:
                        scat_chunk(p, 0, 1).start()
                        scat_chunk(p, 1, 0).start()
                        scat_chunk(p, 1, 1).start()
                else:
                    buf = kbuf if t == 0 else vbuf
                    buf[:] = tmp[slot].astype(jnp.bfloat16)

        def fwd_desc(half, tensor):
            src = relayk if tensor == 0 else relayv
            buf, sems = (kbuf, krecv) if tensor == 0 else (vbuf, vrecv)
            return pltpu.make_async_remote_copy(
                src_ref=src, dst_ref=buf.at[:, pl.ds(HH * half, HH), :],
                send_sem=fwd_send.at[tensor], recv_sem=sems.at[half],
                device_id=(2,), device_id_type=pl.DeviceIdType.MESH)

        def do_relay(half):
            for tensor in (0, 1):
                scat_chunk(2, half, tensor).wait_recv()
                fwd_desc(half, tensor).start()

        @pl.when(my == 1)
        def _():
            do_relay(0)

        @pl.when(my == 3)
        def _():
            do_relay(1)

        x_bf = x_ref[0].astype(jnp.bfloat16)
        q = jnp.dot(x_bf, wq_ref[:].astype(jnp.bfloat16),
                    preferred_element_type=jnp.float32)
        q_bf = (q * SCALE).astype(jnp.bfloat16)
        wo_bf = wo_ref[:].astype(jnp.bfloat16)

        QT = SQ // 4

        def mk_mask(nrows, ncols, row0):
            r = lax.broadcasted_iota(jnp.int32, (nrows, ncols), 0) // BLK
            c = lax.broadcasted_iota(jnp.int32, (nrows, ncols), 1) // BLK
            return c <= (r + row0 // BLK)

        masks = [mk_mask(QT, QT * (qt + 1), QT * qt) for qt in range(4)]

        def attn_q(qt, h):
            r0 = QT * qt
            nkv = QT * (qt + 1)
            qh = q_bf[r0:r0 + QT, DH * h:DH * (h + 1)]
            s = lax.dot_general(
                qh, kbuf[:nkv, h, :], (((1,), (1,)), ((), ())),
                preferred_element_type=jnp.float32)
            w = jnp.exp(jnp.where(masks[qt], s, NEG))
            w = (w / jnp.sum(w, axis=1, keepdims=True)).astype(jnp.bfloat16)
            ctx_buf[r0:r0 + QT, DH * h:DH * (h + 1)] = jnp.dot(
                w, vbuf[:nkv, h, :],
                preferred_element_type=jnp.float32).astype(jnp.bfloat16)

        def wait_half(half):
            @pl.when(my != 0)
            def _():
                scat_chunk(1, half, 0).wait_recv()
                scat_chunk(1, half, 1).wait_recv()

        wait_half(0)
        for qt in range(4):
            for h in range(HH):
                attn_q(qt, h)
        wait_half(1)

        p1 = jnp.bitwise_xor(my, 1)
        p2 = (N_DEV - 1) - my

        def xchg(slot, partner):
            return pltpu.make_async_remote_copy(
                src_ref=sbuf.at[slot], dst_ref=rbuf.at[slot],
                send_sem=ar_send.at[slot], recv_sem=ar_recv.at[slot],
                device_id=(partner,), device_id_type=pl.DeviceIdType.MESH)

        partials = [None] * 4

        def step2_for(qt):
            xchg(qt, p1).wait()
            acc = partials[qt] + rbuf[qt].astype(jnp.float32)
            sbuf[4 + qt] = acc.astype(jnp.bfloat16)
            out_ref[0, QT * qt:QT * (qt + 1)] = acc
            xchg(4 + qt, p2).start()

        for qt in range(4):
            for h in range(HH, HQ):
                attn_q(qt, h)
            r0 = QT * qt
            pq = jnp.dot(ctx_buf[r0:r0 + QT], wo_bf,
                         preferred_element_type=jnp.float32)
            partials[qt] = pq
            sbuf[qt] = pq.astype(jnp.bfloat16)
            xchg(qt, p1).start()
            if qt >= 1:
                step2_for(qt - 1)

        @pl.when(my == 0)
        def _():
            for p in (2, 1, 3):
                for half in (0, 1):
                    scat_chunk(p, half, 0).wait_send()
                    scat_chunk(p, half, 1).wait_send()

        @pl.when(my == 1)
        def _():
            fwd_desc(0, 0).wait_send()
            fwd_desc(0, 1).wait_send()

        @pl.when(my == 3)
        def _():
            fwd_desc(1, 0).wait_send()
            fwd_desc(1, 1).wait_send()

        step2_for(3)
        for qt in range(4):
            xchg(4 + qt, p2).wait()
            out_ref[0, QT * qt:QT * (qt + 1)] = (
                out_ref[0, QT * qt:QT * (qt + 1)]
                + rbuf[4 + qt].astype(jnp.float32))

    return pl.pallas_call(
        body,
        out_shape=jax.ShapeDtypeStruct((1, SQ, 1024), jnp.float32),
        in_specs=[
            pl.BlockSpec(memory_space=pltpu.VMEM),
            pl.BlockSpec(memory_space=pltpu.VMEM),
            pl.BlockSpec(memory_space=pl.ANY),
            pl.BlockSpec(memory_space=pl.ANY),
            pl.BlockSpec(memory_space=pltpu.VMEM),
        ],
        out_specs=pl.BlockSpec(memory_space=pltpu.VMEM),
        scratch_shapes=[
            pltpu.VMEM((2, SQ, HQ, DH), jnp.float32),
            pltpu.VMEM((3, SQ, HQ, DH), jnp.bfloat16),
            pltpu.VMEM((3, SQ, HQ, DH), jnp.bfloat16),
            pltpu.VMEM((SQ, HQ, DH), jnp.bfloat16),
            pltpu.VMEM((SQ, HQ, DH), jnp.bfloat16),
            pltpu.VMEM((SQ, HH, DH), jnp.bfloat16),
            pltpu.VMEM((SQ, HH, DH), jnp.bfloat16),
            pltpu.VMEM((SQ, HQ * DH), jnp.bfloat16),
            pltpu.VMEM((8, SQ // 4, 1024), jnp.bfloat16),
            pltpu.VMEM((8, SQ // 4, 1024), jnp.bfloat16),
            pltpu.SemaphoreType.DMA((2,)),
            pltpu.SemaphoreType.DMA((12,)),
            pltpu.SemaphoreType.DMA((2,)),
            pltpu.SemaphoreType.DMA((2,)),
            pltpu.SemaphoreType.DMA((2,)),
            pltpu.SemaphoreType.DMA((2,)),
            pltpu.SemaphoreType.DMA((8,)),
            pltpu.SemaphoreType.DMA((8,)),
        ],
        compiler_params=pltpu.CompilerParams(
            collective_id=0, vmem_limit_bytes=110 * 1024 * 1024),
    )(x, Wq, K_ext, V_ext, Wo)


# device time: 119461 ns/iter; 2.4088x vs baseline; 1.0280x over previous
import jax
import jax.numpy as jnp
from jax import lax
from jax.experimental import pallas as pl
from jax.experimental.pallas import tpu as pltpu

N_DEV = 4
SQ = 1024
HQ = 8
HH = HQ // 2
DH = 128
BLK = 64
H = SQ // 2
SCALE = 0.08838834764831843
NEG = -1e4


def kernel(x, Wq, K_ext, V_ext, Wo):
    def body(x_ref, wq_ref, k_ref, v_ref, wo_ref, out_ref,
             tmp, kstage, vstage, kbuf, vbuf, relayk, relayv,
             ctx_buf, sbuf, rbuf,
             local_sems, scat_send, krecv, vrecv, relay_recv, fwd_send,
             ar_send, ar_recv):
        my = lax.axis_index("i")

        ORDER = (2, 1, 3, 0)
        SEQ = tuple((p, t) for p in ORDER for t in (0, 1))

        def dma_one(i, slot):
            p, t = SEQ[i]
            src_ref = (k_ref if t == 0 else v_ref).at[0, :, pl.ds(HQ * p, HQ), :]
            return pltpu.make_async_copy(src_ref, tmp.at[slot],
                                         local_sems.at[slot])

        @pl.when(my == 0)
        def _():
            dma_one(0, 0).start()

        barrier_sem = pltpu.get_barrier_semaphore()
        for off in (1, 2, 3):
            pl.semaphore_signal(
                barrier_sem, inc=1,
                device_id=(lax.rem(my + off, N_DEV),),
                device_id_type=pl.DeviceIdType.MESH,
            )
        pl.semaphore_wait(barrier_sem, 3)

        JOF = {2: 0, 1: 1, 3: 2}

        HO = (0, 4, 6)
        HN = (4, 2, 2)

        def scat_chunk(p, c, tensor):
            j = JOF[p]
            stage = kstage if tensor == 0 else vstage
            src = stage.at[j, :, pl.ds(HO[c], HN[c]), :]
            send = scat_send.at[6 * j + 2 * c + tensor]
            if p == 2:
                target = 1 if c == 0 else 3
                rbuf_ = relayk if tensor == 0 else relayv
                dst = rbuf_.at[:, pl.ds(HO[c] % 4, HN[c]), :] if c else rbuf_
                recv = relay_recv.at[2 * c + tensor]
            else:
                target = p
                buf, sems = (kbuf, krecv) if tensor == 0 else (vbuf, vrecv)
                dst = buf.at[:, pl.ds(HO[c], HN[c]), :]
                recv = sems.at[c]
            return pltpu.make_async_remote_copy(
                src_ref=src, dst_ref=dst, send_sem=send, recv_sem=recv,
                device_id=(target,), device_id_type=pl.DeviceIdType.MESH)

        @pl.when(my == 0)
        def _():
            for i, (p, t) in enumerate(SEQ):
                slot = i % 2
                dma_one(i, slot).wait()
                if i + 1 < len(SEQ):
                    dma_one(i + 1, 1 - slot).start()
                if p != 0:
                    stage = kstage if t == 0 else vstage
                    stage[JOF[p]] = tmp[slot].astype(jnp.bfloat16)
                    if t == 0:
                        scat_chunk(p, 0, 0).start()
                    else:
                        scat_chunk(p, 0, 1).start()
                        for c in (1, 2):
                            scat_chunk(p, c, 0).start()
                            scat_chunk(p, c, 1).start()
                else:
                    buf = kbuf if t == 0 else vbuf
                    buf[:] = tmp[slot].astype(jnp.bfloat16)

        def fwd_desc(c, tensor):
            rbuf_ = relayk if tensor == 0 else relayv
            src = rbuf_.at[:, pl.ds(HO[c] % 4, HN[c]), :] if c else rbuf_
            buf, sems = (kbuf, krecv) if tensor == 0 else (vbuf, vrecv)
            return pltpu.make_async_remote_copy(
                src_ref=src, dst_ref=buf.at[:, pl.ds(HO[c], HN[c]), :],
                send_sem=fwd_send.at[2 * c + tensor], recv_sem=sems.at[c],
                device_id=(2,), device_id_type=pl.DeviceIdType.MESH)

        def do_relay(cs):
            for c in cs:
                for tensor in (0, 1):
                    scat_chunk(2, c, tensor).wait_recv()
                    fwd_desc(c, tensor).start()

        @pl.when(my == 1)
        def _():
            do_relay((0,))

        @pl.when(my == 3)
        def _():
            do_relay((1, 2))

        x_bf = x_ref[0].astype(jnp.bfloat16)
        q = jnp.dot(x_bf, wq_ref[:].astype(jnp.bfloat16),
                    preferred_element_type=jnp.float32)
        q_bf = (q * SCALE).astype(jnp.bfloat16)
        wo_bf = wo_ref[:].astype(jnp.bfloat16)

        QT = SQ // 4

        def mk_mask(nrows, ncols, row0):
            r = lax.broadcasted_iota(jnp.int32, (nrows, ncols), 0) // BLK
            c = lax.broadcasted_iota(jnp.int32, (nrows, ncols), 1) // BLK
            return c <= (r + row0 // BLK)

        masks = [mk_mask(QT, QT * (qt + 1), QT * qt) for qt in range(4)]

        def attn_q(qt, h):
            r0 = QT * qt
            nkv = QT * (qt + 1)
            qh = q_bf[r0:r0 + QT, DH * h:DH * (h + 1)]
            s = lax.dot_general(
                qh, kbuf[:nkv, h, :], (((1,), (1,)), ((), ())),
                preferred_element_type=jnp.float32)
            w = jnp.exp(jnp.where(masks[qt], s, NEG))
            w = (w / jnp.sum(w, axis=1, keepdims=True)).astype(jnp.bfloat16)
            ctx_buf[r0:r0 + QT, DH * h:DH * (h + 1)] = jnp.dot(
                w, vbuf[:nkv, h, :],
                preferred_element_type=jnp.float32).astype(jnp.bfloat16)

        def wait_c(c):
            @pl.when(my != 0)
            def _():
                scat_chunk(1, c, 0).wait_recv()
                scat_chunk(1, c, 1).wait_recv()

        wait_c(0)
        for qt in range(4):
            for h in range(HH):
                attn_q(qt, h)
        wait_c(1)
        for qt in range(4):
            for h in (4, 5):
                attn_q(qt, h)
        wait_c(2)

        p1 = jnp.bitwise_xor(my, 1)
        p2 = (N_DEV - 1) - my

        def xchg(slot, partner):
            return pltpu.make_async_remote_copy(
                src_ref=sbuf.at[slot], dst_ref=rbuf.at[slot],
                send_sem=ar_send.at[slot], recv_sem=ar_recv.at[slot],
                device_id=(partner,), device_id_type=pl.DeviceIdType.MESH)

        partials = [None] * 4

        def step2_for(qt):
            xchg(qt, p1).wait()
            acc = partials[qt] + rbuf[qt].astype(jnp.float32)
            sbuf[4 + qt] = acc.astype(jnp.bfloat16)
            out_ref[0, QT * qt:QT * (qt + 1)] = acc
            xchg(4 + qt, p2).start()

        for qt in range(4):
            for h in (6, 7):
                attn_q(qt, h)
            r0 = QT * qt
            pq = jnp.dot(ctx_buf[r0:r0 + QT], wo_bf,
                         preferred_element_type=jnp.float32)
            partials[qt] = pq
            sbuf[qt] = pq.astype(jnp.bfloat16)
            xchg(qt, p1).start()
            if qt >= 1:
                step2_for(qt - 1)

        @pl.when(my == 0)
        def _():
            for p in (2, 1, 3):
                for c in (0, 1, 2):
                    scat_chunk(p, c, 0).wait_send()
                    scat_chunk(p, c, 1).wait_send()

        @pl.when(my == 1)
        def _():
            fwd_desc(0, 0).wait_send()
            fwd_desc(0, 1).wait_send()

        @pl.when(my == 3)
        def _():
            for c in (1, 2):
                fwd_desc(c, 0).wait_send()
                fwd_desc(c, 1).wait_send()

        step2_for(3)
        for qt in range(4):
            xchg(4 + qt, p2).wait()
            out_ref[0, QT * qt:QT * (qt + 1)] = (
                out_ref[0, QT * qt:QT * (qt + 1)]
                + rbuf[4 + qt].astype(jnp.float32))

    return pl.pallas_call(
        body,
        out_shape=jax.ShapeDtypeStruct((1, SQ, 1024), jnp.float32),
        in_specs=[
            pl.BlockSpec(memory_space=pltpu.VMEM),
            pl.BlockSpec(memory_space=pltpu.VMEM),
            pl.BlockSpec(memory_space=pl.ANY),
            pl.BlockSpec(memory_space=pl.ANY),
            pl.BlockSpec(memory_space=pltpu.VMEM),
        ],
        out_specs=pl.BlockSpec(memory_space=pltpu.VMEM),
        scratch_shapes=[
            pltpu.VMEM((2, SQ, HQ, DH), jnp.float32),
            pltpu.VMEM((3, SQ, HQ, DH), jnp.bfloat16),
            pltpu.VMEM((3, SQ, HQ, DH), jnp.bfloat16),
            pltpu.VMEM((SQ, HQ, DH), jnp.bfloat16),
            pltpu.VMEM((SQ, HQ, DH), jnp.bfloat16),
            pltpu.VMEM((SQ, HH, DH), jnp.bfloat16),
            pltpu.VMEM((SQ, HH, DH), jnp.bfloat16),
            pltpu.VMEM((SQ, HQ * DH), jnp.bfloat16),
            pltpu.VMEM((8, SQ // 4, 1024), jnp.bfloat16),
            pltpu.VMEM((8, SQ // 4, 1024), jnp.bfloat16),
            pltpu.SemaphoreType.DMA((2,)),
            pltpu.SemaphoreType.DMA((18,)),
            pltpu.SemaphoreType.DMA((3,)),
            pltpu.SemaphoreType.DMA((3,)),
            pltpu.SemaphoreType.DMA((6,)),
            pltpu.SemaphoreType.DMA((6,)),
            pltpu.SemaphoreType.DMA((8,)),
            pltpu.SemaphoreType.DMA((8,)),
        ],
        compiler_params=pltpu.CompilerParams(
            collective_id=0, vmem_limit_bytes=110 * 1024 * 1024),
    )(x, Wq, K_ext, V_ext, Wo)
